# baseline (speedup 1.0000x reference)
import functools

import jax
import jax.numpy as jnp
from jax import lax
from jax.experimental import pallas as pl
from jax.experimental.pallas import tpu as pltpu

_BLK = 1024


def _nms_body(bt_ref, brow_ref, out_ref,
              s_ref, keep_ref, kbc_ref, *, blk):
    i = pl.program_id(0)
    j = pl.program_id(1)
    ibase = pl.multiple_of(i * blk, blk)
    jbase = pl.multiple_of(j * blk, blk)

    @pl.when((i == 0) & (j == 0))
    def _():
        keep_ref[...] = jnp.ones_like(keep_ref)

    @pl.when(j == i)
    def _diagonal():
        x1 = brow_ref[:, 0:1]
        y1 = brow_ref[:, 1:2]
        x2 = brow_ref[:, 2:3]
        y2 = brow_ref[:, 3:4]
        ar = brow_ref[:, 5:6]
        cx1 = bt_ref[0:1, pl.ds(ibase, blk)]
        cy1 = bt_ref[1:2, pl.ds(ibase, blk)]
        cx2 = bt_ref[2:3, pl.ds(ibase, blk)]
        cy2 = bt_ref[3:4, pl.ds(ibase, blk)]
        ac = bt_ref[4:5, pl.ds(ibase, blk)]
        iw = jnp.maximum(jnp.minimum(x2, cx2) - jnp.maximum(x1, cx1), 0.0)
        ih = jnp.minimum(y2, cy2) - jnp.maximum(y1, cy1)
        overlap = 3.0 * (iw * ih) > ar + ac
        tri_r = lax.broadcasted_iota(jnp.int32, (blk, blk), 0)
        tri_c = lax.broadcasted_iota(jnp.int32, (blk, blk), 1)
        s_ref[...] = jnp.where((tri_c > tri_r) & overlap, 1.0, 0.0)

        keep0 = keep_ref[:, pl.ds(ibase, blk)]

        def _step(kb):
            cnt = jnp.dot(kb, s_ref[...],
                          preferred_element_type=jnp.float32)
            return keep0 * jnp.where(cnt > 0.5, 0.0, 1.0)

        def _cond(carry):
            return carry[1]

        def _sweep(carry):
            kb, _ = carry
            kb1 = _step(_step(_step(kb)))
            kb2 = _step(kb1)
            changed = jnp.any(kb2 != kb1)
            return (kb2, changed)

        kb, _ = lax.while_loop(_cond, _sweep, (keep0, True))
        keep_ref[:, pl.ds(ibase, blk)] = kb
        kbc = kb.T
        kbc_ref[...] = kbc

        out_ref[...] = brow_ref[:, 0:5] * kbc

    @pl.when(j > i)
    def _tail():
        kbc = kbc_ref[...]
        x1 = jnp.where(kbc > 0.0, brow_ref[:, 0:1], 3.0e4)
        y1 = brow_ref[:, 1:2]
        x2 = brow_ref[:, 2:3]
        y2 = brow_ref[:, 3:4]
        ar = brow_ref[:, 5:6]
        cx1 = bt_ref[0:1, pl.ds(jbase, blk)]
        cy1 = bt_ref[1:2, pl.ds(jbase, blk)]
        cx2 = bt_ref[2:3, pl.ds(jbase, blk)]
        cy2 = bt_ref[3:4, pl.ds(jbase, blk)]
        ac = bt_ref[4:5, pl.ds(jbase, blk)]
        iw = jnp.maximum(jnp.minimum(x2, cx2) - jnp.maximum(x1, cx1), 0.0)
        ih = jnp.minimum(y2, cy2) - jnp.maximum(y1, cy1)
        overlap = 3.0 * (iw * ih) > ar + ac
        sup = jnp.any(overlap, axis=0, keepdims=True)
        keep_ref[:, pl.ds(jbase, blk)] = jnp.where(
            sup, 0.0, keep_ref[:, pl.ds(jbase, blk)])


def kernel(boxes, scores):
    n = boxes.shape[0]
    blk = _BLK
    nb = -(-n // blk)
    n_pad = nb * blk
    pad = n_pad - n

    neg, x1, y1, x2, y2 = lax.sort(
        (-scores, boxes[:, 0], boxes[:, 1], boxes[:, 2], boxes[:, 3]),
        num_keys=1)
    s = -neg
    area = (x2 - x1) * (y2 - y1)
    z = jnp.zeros((pad,), jnp.float32)
    x1 = jnp.concatenate([x1, z])
    y1 = jnp.concatenate([y1, z])
    x2 = jnp.concatenate([x2, z])
    y2 = jnp.concatenate([y2, z])
    s = jnp.concatenate([s, z])
    area = jnp.concatenate([area, z])
    bt = jnp.stack([x1, y1, x2, y2, area], axis=0)
    rows = jnp.stack([x1, y1, x2, y2, s, area], axis=1)

    out = pl.pallas_call(
        functools.partial(_nms_body, blk=blk),
        grid=(nb, nb),
        in_specs=[
            pl.BlockSpec((5, n_pad), lambda i, j: (0, 0)),
            pl.BlockSpec((blk, 6), lambda i, j: (i, 0)),
        ],
        out_specs=pl.BlockSpec((blk, 5), lambda i, j: (i, 0)),
        out_shape=jax.ShapeDtypeStruct((n_pad, 5), jnp.float32),
        scratch_shapes=[
            pltpu.VMEM((blk, blk), jnp.float32),
            pltpu.VMEM((1, n_pad), jnp.float32),
            pltpu.VMEM((blk, 1), jnp.float32),
        ],
    )(bt, rows)

    return out[:n]


# speedup vs baseline: 1.0458x; 1.0458x over previous
import functools

import jax
import jax.numpy as jnp
from jax import lax
from jax.experimental import pallas as pl
from jax.experimental.pallas import tpu as pltpu

_BLK = 1024


def _nms_body(bt_ref, brow_ref, out_ref,
              s_ref, keep_ref, kbc_ref, *, blk):
    i = pl.program_id(0)
    j = pl.program_id(1)
    ibase = pl.multiple_of(i * blk, blk)
    jbase = pl.multiple_of(j * blk, blk)

    @pl.when((i == 0) & (j == 0))
    def _():
        keep_ref[...] = jnp.ones_like(keep_ref)

    @pl.when(j == i)
    def _diagonal():
        x1 = brow_ref[:, 0:1]
        y1 = brow_ref[:, 1:2]
        x2 = brow_ref[:, 2:3]
        y2 = brow_ref[:, 3:4]
        ar = brow_ref[:, 5:6]
        cx1 = bt_ref[0:1, pl.ds(ibase, blk)]
        cy1 = bt_ref[1:2, pl.ds(ibase, blk)]
        cx2 = bt_ref[2:3, pl.ds(ibase, blk)]
        cy2 = bt_ref[3:4, pl.ds(ibase, blk)]
        ac = bt_ref[4:5, pl.ds(ibase, blk)]
        iw = jnp.maximum(jnp.minimum(x2, cx2) - jnp.maximum(x1, cx1), 0.0)
        ih = jnp.minimum(y2, cy2) - jnp.maximum(y1, cy1)
        overlap = 3.0 * (iw * ih) > ar + ac
        tri_r = lax.broadcasted_iota(jnp.int32, (blk, blk), 0)
        tri_c = lax.broadcasted_iota(jnp.int32, (blk, blk), 1)
        s_ref[...] = jnp.where((tri_c > tri_r) & overlap, 1.0, 0.0)

        keep0 = keep_ref[:, pl.ds(ibase, blk)]

        def _step(kb):
            cnt = jnp.dot(kb, s_ref[...],
                          preferred_element_type=jnp.float32)
            return keep0 * jnp.where(cnt > 0.5, 0.0, 1.0)

        def _cond(carry):
            return carry[1]

        def _sweep(carry):
            kb, _ = carry
            kb1 = _step(kb)
            kb2 = _step(kb1)
            changed = jnp.any(kb2 != kb1)
            return (kb2, changed)

        kb, _ = lax.while_loop(_cond, _sweep, (keep0, True))
        keep_ref[:, pl.ds(ibase, blk)] = kb
        kbc = kb.T
        kbc_ref[...] = kbc

        out_ref[...] = brow_ref[:, 0:5] * kbc

    @pl.when(j > i)
    def _tail():
        kbc = kbc_ref[...]
        x1 = jnp.where(kbc > 0.0, brow_ref[:, 0:1], 3.0e4)
        y1 = brow_ref[:, 1:2]
        x2 = brow_ref[:, 2:3]
        y2 = brow_ref[:, 3:4]
        ar = brow_ref[:, 5:6]
        cx1 = bt_ref[0:1, pl.ds(jbase, blk)]
        cy1 = bt_ref[1:2, pl.ds(jbase, blk)]
        cx2 = bt_ref[2:3, pl.ds(jbase, blk)]
        cy2 = bt_ref[3:4, pl.ds(jbase, blk)]
        ac = bt_ref[4:5, pl.ds(jbase, blk)]
        iw = jnp.maximum(jnp.minimum(x2, cx2) - jnp.maximum(x1, cx1), 0.0)
        ih = jnp.minimum(y2, cy2) - jnp.maximum(y1, cy1)
        overlap = 3.0 * (iw * ih) > ar + ac
        sup = jnp.any(overlap, axis=0, keepdims=True)
        keep_ref[:, pl.ds(jbase, blk)] = jnp.where(
            sup, 0.0, keep_ref[:, pl.ds(jbase, blk)])


def kernel(boxes, scores):
    n = boxes.shape[0]
    blk = _BLK
    nb = -(-n // blk)
    n_pad = nb * blk
    pad = n_pad - n

    neg, x1, y1, x2, y2 = lax.sort(
        (-scores, boxes[:, 0], boxes[:, 1], boxes[:, 2], boxes[:, 3]),
        num_keys=1)
    s = -neg
    area = (x2 - x1) * (y2 - y1)
    z = jnp.zeros((pad,), jnp.float32)
    x1 = jnp.concatenate([x1, z])
    y1 = jnp.concatenate([y1, z])
    x2 = jnp.concatenate([x2, z])
    y2 = jnp.concatenate([y2, z])
    s = jnp.concatenate([s, z])
    area = jnp.concatenate([area, z])
    bt = jnp.stack([x1, y1, x2, y2, area], axis=0)
    rows = jnp.stack([x1, y1, x2, y2, s, area], axis=1)

    out = pl.pallas_call(
        functools.partial(_nms_body, blk=blk),
        grid=(nb, nb),
        in_specs=[
            pl.BlockSpec((5, n_pad), lambda i, j: (0, 0)),
            pl.BlockSpec((blk, 6), lambda i, j: (i, 0)),
        ],
        out_specs=pl.BlockSpec((blk, 5), lambda i, j: (i, 0)),
        out_shape=jax.ShapeDtypeStruct((n_pad, 5), jnp.float32),
        scratch_shapes=[
            pltpu.VMEM((blk, blk), jnp.float32),
            pltpu.VMEM((1, n_pad), jnp.float32),
            pltpu.VMEM((blk, 1), jnp.float32),
        ],
    )(bt, rows)

    return out[:n]
